# baseline (speedup 1.0000x reference)
import functools

import jax
import jax.numpy as jnp
from jax import lax
from jax.experimental import pallas as pl
from jax.experimental.pallas import tpu as pltpu
from jax.experimental.pallas import tpu_sc as plsc

N_ENT = 100000
RANK = 128
BATCH = 1024

NC, NS = 2, 16
NW = NC * NS
B_PER_W = BATCH // NW

BN = 2048


def _sc_gather_body(x0_hbm, x1_hbm, x2_hbm, lhs_hbm, rel_hbm, rhs_hbm,
                    out_l, out_r, out_o,
                    idx0_v, idx1_v, idx2_v, buf_l, buf_r, buf_o, sem):
    wid = lax.axis_index("s") * NC + lax.axis_index("c")
    base = wid * B_PER_W
    pltpu.sync_copy(x0_hbm.at[pl.ds(base, B_PER_W)], idx0_v)
    pltpu.sync_copy(x1_hbm.at[pl.ds(base, B_PER_W)], idx1_v)
    pltpu.sync_copy(x2_hbm.at[pl.ds(base, B_PER_W)], idx2_v)
    cl = pltpu.async_copy(lhs_hbm.at[idx0_v], buf_l, sem)
    cr = pltpu.async_copy(rel_hbm.at[idx1_v], buf_r, sem)
    co = pltpu.async_copy(rhs_hbm.at[idx2_v], buf_o, sem)
    cl.wait()
    cr.wait()
    co.wait()
    pltpu.sync_copy(buf_l, out_l.at[pl.ds(base, B_PER_W)])
    pltpu.sync_copy(buf_r, out_r.at[pl.ds(base, B_PER_W)])
    pltpu.sync_copy(buf_o, out_o.at[pl.ds(base, B_PER_W)])


@functools.cache
def _sc_gather():
    return functools.partial(
        pl.kernel,
        out_type=[jax.ShapeDtypeStruct((BATCH, RANK), jnp.float32)] * 3,
        mesh=plsc.VectorSubcoreMesh(core_axis_name="c", subcore_axis_name="s"),
        scratch_types=[
            pltpu.VMEM((B_PER_W,), jnp.int32),
            pltpu.VMEM((B_PER_W,), jnp.int32),
            pltpu.VMEM((B_PER_W,), jnp.int32),
            pltpu.VMEM((B_PER_W, RANK), jnp.float32),
            pltpu.VMEM((B_PER_W, RANK), jnp.float32),
            pltpu.VMEM((B_PER_W, RANK), jnp.float32),
            pltpu.SemaphoreType.DMA,
        ],
    )(_sc_gather_body)


def _mm_body(lhs_ref, rel_ref, rhs_ref, out_ref, lr_ref):
    @pl.when(pl.program_id(0) == 0)
    def _():
        lr_ref[...] = (lhs_ref[...] * rel_ref[...]).astype(jnp.bfloat16)

    out_ref[...] = lax.dot_general(
        lr_ref[...], rhs_ref[...].astype(jnp.bfloat16),
        (((1,), (1,)), ((), ())), preferred_element_type=jnp.float32)


def _matmul(lhs, rel, rhs_w):
    return pl.pallas_call(
        _mm_body,
        grid=(pl.cdiv(N_ENT, BN),),
        in_specs=[
            pl.BlockSpec((BATCH, RANK), lambda i: (0, 0)),
            pl.BlockSpec((BATCH, RANK), lambda i: (0, 0)),
            pl.BlockSpec((BN, RANK), lambda i: (i, 0)),
        ],
        out_specs=pl.BlockSpec((BATCH, BN), lambda i: (0, i)),
        out_shape=jax.ShapeDtypeStruct((BATCH, N_ENT), jnp.float32),
        scratch_shapes=[pltpu.VMEM((BATCH, RANK), jnp.bfloat16)],
        compiler_params=pltpu.CompilerParams(
            dimension_semantics=("arbitrary",)),
    )(lhs, rel, rhs_w)


def kernel(x, lhs_w, rel_w, rhs_w):
    xi = x.astype(jnp.int32)
    x0 = jnp.ravel(xi[:, 0])
    x1 = jnp.ravel(xi[:, 1])
    x2 = jnp.ravel(xi[:, 2])
    lhs, rel, rhs = _sc_gather()(x0, x1, x2, lhs_w, rel_w, rhs_w)
    rhs_scores = _matmul(lhs, rel, rhs_w)
    return (rhs_scores, (lhs, rel, rhs))


# speedup vs baseline: 1.0023x; 1.0023x over previous
import functools

import jax
import jax.numpy as jnp
from jax import lax
from jax.experimental import pallas as pl
from jax.experimental.pallas import tpu as pltpu
from jax.experimental.pallas import tpu_sc as plsc

N_ENT = 100000
RANK = 128
BATCH = 1024

NC, NS = 2, 16
NW = NC * NS
B_PER_W = BATCH // NW

BN = 4096


def _sc_gather_body(x0_hbm, x1_hbm, x2_hbm, lhs_hbm, rel_hbm, rhs_hbm,
                    out_l, out_r, out_o,
                    idx0_v, idx1_v, idx2_v, buf_l, buf_r, buf_o, sem):
    wid = lax.axis_index("s") * NC + lax.axis_index("c")
    base = wid * B_PER_W
    pltpu.sync_copy(x0_hbm.at[pl.ds(base, B_PER_W)], idx0_v)
    pltpu.sync_copy(x1_hbm.at[pl.ds(base, B_PER_W)], idx1_v)
    pltpu.sync_copy(x2_hbm.at[pl.ds(base, B_PER_W)], idx2_v)
    cl = pltpu.async_copy(lhs_hbm.at[idx0_v], buf_l, sem)
    cr = pltpu.async_copy(rel_hbm.at[idx1_v], buf_r, sem)
    co = pltpu.async_copy(rhs_hbm.at[idx2_v], buf_o, sem)
    cl.wait()
    cr.wait()
    co.wait()
    pltpu.sync_copy(buf_l, out_l.at[pl.ds(base, B_PER_W)])
    pltpu.sync_copy(buf_r, out_r.at[pl.ds(base, B_PER_W)])
    pltpu.sync_copy(buf_o, out_o.at[pl.ds(base, B_PER_W)])


@functools.cache
def _sc_gather():
    return functools.partial(
        pl.kernel,
        out_type=[jax.ShapeDtypeStruct((BATCH, RANK), jnp.float32)] * 3,
        mesh=plsc.VectorSubcoreMesh(core_axis_name="c", subcore_axis_name="s"),
        scratch_types=[
            pltpu.VMEM((B_PER_W,), jnp.int32),
            pltpu.VMEM((B_PER_W,), jnp.int32),
            pltpu.VMEM((B_PER_W,), jnp.int32),
            pltpu.VMEM((B_PER_W, RANK), jnp.float32),
            pltpu.VMEM((B_PER_W, RANK), jnp.float32),
            pltpu.VMEM((B_PER_W, RANK), jnp.float32),
            pltpu.SemaphoreType.DMA,
        ],
    )(_sc_gather_body)


def _mm_body(lhs_ref, rel_ref, rhs_ref, out_ref, lr_ref):
    @pl.when(pl.program_id(0) == 0)
    def _():
        lr_ref[...] = (lhs_ref[...] * rel_ref[...]).astype(jnp.bfloat16)

    out_ref[...] = lax.dot_general(
        lr_ref[...], rhs_ref[...].astype(jnp.bfloat16),
        (((1,), (1,)), ((), ())), preferred_element_type=jnp.float32)


def _matmul(lhs, rel, rhs_w):
    return pl.pallas_call(
        _mm_body,
        grid=(pl.cdiv(N_ENT, BN),),
        in_specs=[
            pl.BlockSpec((BATCH, RANK), lambda i: (0, 0)),
            pl.BlockSpec((BATCH, RANK), lambda i: (0, 0)),
            pl.BlockSpec((BN, RANK), lambda i: (i, 0)),
        ],
        out_specs=pl.BlockSpec((BATCH, BN), lambda i: (0, i)),
        out_shape=jax.ShapeDtypeStruct((BATCH, N_ENT), jnp.float32),
        scratch_shapes=[pltpu.VMEM((BATCH, RANK), jnp.bfloat16)],
        compiler_params=pltpu.CompilerParams(
            dimension_semantics=("arbitrary",)),
    )(lhs, rel, rhs_w)


def kernel(x, lhs_w, rel_w, rhs_w):
    xi = x.astype(jnp.int32)
    x0 = jnp.ravel(xi[:, 0])
    x1 = jnp.ravel(xi[:, 1])
    x2 = jnp.ravel(xi[:, 2])
    lhs, rel, rhs = _sc_gather()(x0, x1, x2, lhs_w, rel_w, rhs_w)
    rhs_scores = _matmul(lhs, rel, rhs_w)
    return (rhs_scores, (lhs, rel, rhs))
